# baseline (speedup 1.0000x reference)
import functools

import jax
import jax.numpy as jnp
from jax import lax
from jax.experimental import pallas as pl
from jax.experimental.pallas import tpu as pltpu
from jax.experimental.pallas import tpu_sc as plsc

VOCAB = 100000
DIM = 64
B = 1024
SENT_COUNT = 10
SENT_LENGTH = 20

N_UR = B * SENT_COUNT * SENT_LENGTH
N_UI = B * SENT_LENGTH

NC = 2
NS = 16
NW = NC * NS
BPW = B // NW

GROUP = 320
SPG = GROUP // SENT_LENGTH

PW_UR = N_UR // NW
PW_UI = N_UI // NW
NG_UR = PW_UR // GROUP
NG_UI = PW_UI // GROUP
NG = 2 * NG_UR + NG_UI


def _emb_kernel(ur_idx, ir_idx, ui_idx, table,
                out_ur, out_ir, out_ui,
                urv, irv, uiv, rows_v, gsem0, gsem1, ssem0, ssem1):
    wid = lax.axis_index("s") * NC + lax.axis_index("c")
    gsems = (gsem0, gsem1)
    ssems = (ssem0, ssem1)

    pltpu.sync_copy(ur_idx.at[pl.ds(wid * BPW, BPW)], urv)
    pltpu.sync_copy(ir_idx.at[pl.ds(wid * BPW, BPW)], irv)
    pltpu.sync_copy(ui_idx.at[pl.ds(wid * BPW, BPW)], uiv)

    def _fire(idx_of, g, p):
        for j in range(SPG):
            idx = idx_of(g * SPG + j)
            pltpu.async_copy(
                table.at[idx],
                rows_v.at[p, pl.ds(j * SENT_LENGTH, SENT_LENGTH)],
                gsems[p])

    def fire_ur(g, p):
        _fire(lambda s: urv.at[s // SENT_COUNT, s % SENT_COUNT], g, p)

    def fire_ir(g, p):
        _fire(lambda s: irv.at[s // SENT_COUNT, s % SENT_COUNT], g, p)

    def fire_ui(g, p):
        _fire(lambda s: uiv.at[s], g, p)

    def drain_gathers(p):
        pltpu.make_async_copy(table.at[pl.ds(0, GROUP)],
                              rows_v.at[p], gsems[p]).wait()

    fire_ur(0, 0)
    fire_ur(1, 1)

    def store_group(p, out_hbm, g):
        pltpu.async_copy(
            rows_v.at[p],
            out_hbm.at[pl.ds(wid * PW_UR + g * GROUP, GROUP)],
            ssems[p]).wait()

    def make_body(out_hbm, fire):
        def body(i, carry):
            s = 2 * i
            for p in (0, 1):
                g = s + p
                drain_gathers(p)
                store_group(p, out_hbm, g)
                fire(g + 2, p)
            return carry
        return body

    lax.fori_loop(0, NG_UR // 2 - 1, make_body(out_ur, fire_ur), 0)
    for p in (0, 1):
        drain_gathers(p)
        store_group(p, out_ur, NG_UR - 2 + p)
        fire_ir(p, p)
    lax.fori_loop(0, NG_UR // 2 - 1, make_body(out_ir, fire_ir), 0)
    for p in (0, 1):
        drain_gathers(p)
        store_group(p, out_ir, NG_UR - 2 + p)
        fire_ui(p, p)
    for p in (0, 1):
        drain_gathers(p)
        pltpu.async_copy(
            rows_v.at[p],
            out_ui.at[pl.ds(wid * PW_UI + p * GROUP, GROUP)],
            ssems[p]).wait()


@jax.jit
def _run(ur3, ir3, ui2, word_emb):
    mesh = plsc.VectorSubcoreMesh(core_axis_name="c", subcore_axis_name="s")
    return pl.kernel(
        _emb_kernel,
        mesh=mesh,
        out_type=[
            jax.ShapeDtypeStruct((N_UR, DIM), jnp.float32),
            jax.ShapeDtypeStruct((N_UR, DIM), jnp.float32),
            jax.ShapeDtypeStruct((N_UI, DIM), jnp.float32),
        ],
        scratch_types=[
            pltpu.VMEM((BPW, SENT_COUNT, SENT_LENGTH), jnp.int32),
            pltpu.VMEM((BPW, SENT_COUNT, SENT_LENGTH), jnp.int32),
            pltpu.VMEM((BPW, SENT_LENGTH), jnp.int32),
            pltpu.VMEM((2, GROUP, DIM), jnp.float32),
            pltpu.SemaphoreType.DMA,
            pltpu.SemaphoreType.DMA,
            pltpu.SemaphoreType.DMA,
            pltpu.SemaphoreType.DMA,
        ],
        compiler_params=pltpu.CompilerParams(use_tc_tiling_on_sc=False),
    )(ur3, ir3, ui2, word_emb)


def kernel(user_reviews, item_reviews, ui_review, word_emb):
    out_ur, out_ir, out_ui = _run(user_reviews, item_reviews, ui_review,
                                  word_emb)
    return (
        out_ur.reshape(B, SENT_COUNT * SENT_LENGTH, DIM),
        out_ir.reshape(B, SENT_COUNT * SENT_LENGTH, DIM),
        out_ui.reshape(B, SENT_LENGTH, DIM),
    )


# speedup vs baseline: 1.0458x; 1.0458x over previous
import functools

import jax
import jax.numpy as jnp
from jax import lax
from jax.experimental import pallas as pl
from jax.experimental.pallas import tpu as pltpu
from jax.experimental.pallas import tpu_sc as plsc

VOCAB = 100000
DIM = 64
B = 1024
SENT_COUNT = 10
SENT_LENGTH = 20
SENT = SENT_COUNT * SENT_LENGTH

N_UR = B * SENT
N_UI = B * SENT_LENGTH

NC = 2
NS = 16
NW = NC * NS
BPW = B // NW

GB = 4
GROUP = GB * SENT
SPLITS = ((0, 80), (80, 80), (160, 40))

PW_UR = N_UR // NW
PW_UI = N_UI // NW
NG_R = BPW // GB


def _emb_kernel(idx_hbm, table,
                out_ur, out_ir, out_ui,
                idx_v, rows_v, gsem0, gsem1, ssem0, ssem1):
    wid = lax.axis_index("s") * NC + lax.axis_index("c")
    gsems = (gsem0, gsem1)
    ssems = (ssem0, ssem1)

    pltpu.sync_copy(idx_hbm.at[pl.ds(wid * PW_UR, PW_UR)],
                    idx_v.at[pl.ds(0, PW_UR)])
    pltpu.sync_copy(idx_hbm.at[pl.ds(NW * PW_UR + wid * PW_UR, PW_UR)],
                    idx_v.at[pl.ds(PW_UR, PW_UR)])
    pltpu.sync_copy(idx_hbm.at[pl.ds(2 * NW * PW_UR + wid * PW_UI, PW_UI)],
                    idx_v.at[pl.ds(2 * PW_UR, PW_UI)])

    def _fire(base, g, p):
        for b in range(GB):
            for off, ln in SPLITS:
                pltpu.async_copy(
                    table.at[idx_v.at[pl.ds(base + g * GROUP + b * SENT
                                            + off, ln)]],
                    rows_v.at[p, b, pl.ds(off, ln)],
                    gsems[p])

    def fire_ur(g, p):
        _fire(0, g, p)

    def fire_ir(g, p):
        _fire(PW_UR, g, p)

    def fire_ui(p):
        for b in range(3):
            for off, ln in SPLITS:
                pltpu.async_copy(
                    table.at[idx_v.at[pl.ds(2 * PW_UR + b * SENT + off, ln)]],
                    rows_v.at[p, b, pl.ds(off, ln)],
                    gsems[p])
        pltpu.async_copy(
            table.at[idx_v.at[pl.ds(2 * PW_UR + 3 * SENT, 40)]],
            rows_v.at[p, 3, pl.ds(0, 40)],
            gsems[p])

    def drain_group(p):
        pltpu.make_async_copy(out_ur.at[pl.ds(0, GB)],
                              rows_v.at[p], gsems[p]).wait()

    def store_group(p, out_hbm, g):
        pltpu.async_copy(
            rows_v.at[p],
            out_hbm.at[pl.ds(wid * BPW + g * GB, GB)],
            ssems[p]).wait()

    fire_ur(0, 0)
    fire_ur(1, 1)

    def make_body(out_hbm, fire):
        def body(i, carry):
            for p in (0, 1):
                g = 2 * i + p
                drain_group(p)
                store_group(p, out_hbm, g)
                fire(g + 2, p)
            return carry
        return body

    lax.fori_loop(0, NG_R // 2 - 1, make_body(out_ur, fire_ur), 0)
    for p in (0, 1):
        drain_group(p)
        store_group(p, out_ur, NG_R - 2 + p)
        fire_ir(p, p)
    lax.fori_loop(0, NG_R // 2 - 1, make_body(out_ir, fire_ir), 0)
    for p in (0, 1):
        drain_group(p)
        store_group(p, out_ir, NG_R - 2 + p)
        if p == 0:
            fire_ui(0)
    pltpu.make_async_copy(out_ur.at[pl.ds(0, 3)],
                          rows_v.at[0, pl.ds(0, 3)], gsem0).wait()
    pltpu.make_async_copy(out_ui.at[pl.ds(0, 40)],
                          rows_v.at[0, 3, pl.ds(0, 40)], gsem0).wait()
    for b in range(3):
        pltpu.async_copy(rows_v.at[0, b],
                         out_ui.at[pl.ds(wid * PW_UI + b * SENT, SENT)],
                         ssem0).wait()
    pltpu.async_copy(rows_v.at[0, 3, pl.ds(0, 40)],
                     out_ui.at[pl.ds(wid * PW_UI + 3 * SENT, 40)],
                     ssem0).wait()


@jax.jit
def _run(idx_all, word_emb):
    mesh = plsc.VectorSubcoreMesh(core_axis_name="c", subcore_axis_name="s")
    return pl.kernel(
        _emb_kernel,
        mesh=mesh,
        out_type=[
            jax.ShapeDtypeStruct((B, SENT, DIM), jnp.float32),
            jax.ShapeDtypeStruct((B, SENT, DIM), jnp.float32),
            jax.ShapeDtypeStruct((N_UI, DIM), jnp.float32),
        ],
        scratch_types=[
            pltpu.VMEM((2 * PW_UR + PW_UI,), jnp.int32),
            pltpu.VMEM((2, GB, SENT, DIM), jnp.float32),
            pltpu.SemaphoreType.DMA,
            pltpu.SemaphoreType.DMA,
            pltpu.SemaphoreType.DMA,
            pltpu.SemaphoreType.DMA,
        ],
        compiler_params=pltpu.CompilerParams(use_tc_tiling_on_sc=False),
    )(idx_all, word_emb)


def kernel(user_reviews, item_reviews, ui_review, word_emb):
    idx_all = jnp.concatenate([
        user_reviews.reshape(-1),
        item_reviews.reshape(-1),
        ui_review.reshape(-1),
    ])
    out_ur, out_ir, out_ui = _run(idx_all, word_emb)
    return (out_ur, out_ir, out_ui.reshape(B, SENT_LENGTH, DIM))


# speedup vs baseline: 1.0980x; 1.0500x over previous
import functools

import jax
import jax.numpy as jnp
from jax import lax
from jax.experimental import pallas as pl
from jax.experimental.pallas import tpu as pltpu
from jax.experimental.pallas import tpu_sc as plsc

VOCAB = 100000
DIM = 64
B = 1024
SENT_COUNT = 10
SENT_LENGTH = 20

N_UR = B * SENT_COUNT * SENT_LENGTH
N_UI = B * SENT_LENGTH

NC = 2
NS = 16
NW = NC * NS

CHUNK = 128
K = 5
GROUP = K * CHUNK

CH_UR = N_UR // NW // CHUNK
CH_UI = N_UI // NW // CHUNK
NG_UR = CH_UR // K

PW_UR = N_UR // NW
PW_UI = N_UI // NW


def _pipeline(table, idx_v, rows_v, gsems, ssems, wid, stores, ng):
    def fire_group(g, p):
        for b in range(K):
            pltpu.async_copy(
                table.at[idx_v.at[g * K + b]],
                rows_v.at[p, pl.ds(b * CHUNK, CHUNK)],
                gsems[p])

    def drain_gathers(p):
        pltpu.make_async_copy(table.at[pl.ds(0, GROUP)],
                              rows_v.at[p], gsems[p]).wait()

    fire_group(0, 0)
    if ng > 1:
        fire_group(1, 1)

    def body(i, carry):
        for p in (0, 1):
            g = 2 * i + p
            drain_gathers(p)
            stores(g, p)
            fire_group(g + 2, p)
        return carry

    lax.fori_loop(0, (ng - 2) // 2, body, 0)
    for p in (0, 1):
        g = ng - 2 + p
        drain_gathers(p)
        stores(g, p)
    return drain_gathers


def _emb_ur(idx_hbm, table, out_ur,
            idx_v, rows_v, gsem0, gsem1, ssem0, ssem1):
    wid = lax.axis_index("s") * NC + lax.axis_index("c")
    pltpu.sync_copy(idx_hbm.at[pl.ds(wid * CH_UR, CH_UR)], idx_v)

    def stores(g, p):
        pltpu.async_copy(
            rows_v.at[p],
            out_ur.at[pl.ds(wid * PW_UR + g * GROUP, GROUP)],
            (ssem0, ssem1)[p]).wait()

    _pipeline(table, idx_v, rows_v, (gsem0, gsem1), (ssem0, ssem1),
              wid, stores, NG_UR)


def _emb_ir_ui(idx_hbm, table, out_ir, out_ui,
               idx_v, rows_v, gsem0, gsem1, ssem0, ssem1):
    wid = lax.axis_index("s") * NC + lax.axis_index("c")
    pltpu.sync_copy(idx_hbm.at[pl.ds(wid * CH_UR, CH_UR)],
                    idx_v.at[pl.ds(0, CH_UR)])
    pltpu.sync_copy(idx_hbm.at[pl.ds(NW * CH_UR + wid * CH_UI, CH_UI)],
                    idx_v.at[pl.ds(CH_UR, CH_UI)])

    gsems = (gsem0, gsem1)

    def fire_ui(p):
        for b in range(K):
            pltpu.async_copy(
                table.at[idx_v.at[NG_UR * K + b]],
                rows_v.at[p, pl.ds(b * CHUNK, CHUNK)],
                gsems[p])

    def stores(g, p):
        pltpu.async_copy(
            rows_v.at[p],
            out_ir.at[pl.ds(wid * PW_UR + g * GROUP, GROUP)],
            (ssem0, ssem1)[p]).wait()

    drain = _pipeline(table, idx_v, rows_v, gsems, (ssem0, ssem1),
                      wid, stores, NG_UR)
    fire_ui(0)
    drain(0)
    pltpu.sync_copy(rows_v.at[0], out_ui.at[pl.ds(wid * PW_UI, PW_UI)])


@jax.jit
def _run(idx_ur, idx_irui, word_emb):
    mesh = plsc.VectorSubcoreMesh(core_axis_name="c", subcore_axis_name="s")
    common = dict(
        mesh=mesh,
        compiler_params=pltpu.CompilerParams(use_tc_tiling_on_sc=False),
    )
    out_ur = pl.kernel(
        _emb_ur,
        out_type=jax.ShapeDtypeStruct((N_UR, DIM), jnp.float32),
        scratch_types=[
            pltpu.VMEM((CH_UR, CHUNK), jnp.int32),
            pltpu.VMEM((2, GROUP, DIM), jnp.float32),
            pltpu.SemaphoreType.DMA,
            pltpu.SemaphoreType.DMA,
            pltpu.SemaphoreType.DMA,
            pltpu.SemaphoreType.DMA,
        ],
        **common,
    )(idx_ur, word_emb)
    out_ir, out_ui = pl.kernel(
        _emb_ir_ui,
        out_type=[
            jax.ShapeDtypeStruct((N_UR, DIM), jnp.float32),
            jax.ShapeDtypeStruct((N_UI, DIM), jnp.float32),
        ],
        scratch_types=[
            pltpu.VMEM((CH_UR + CH_UI, CHUNK), jnp.int32),
            pltpu.VMEM((2, GROUP, DIM), jnp.float32),
            pltpu.SemaphoreType.DMA,
            pltpu.SemaphoreType.DMA,
            pltpu.SemaphoreType.DMA,
            pltpu.SemaphoreType.DMA,
        ],
        **common,
    )(idx_irui, word_emb)
    return out_ur, out_ir, out_ui


def kernel(user_reviews, item_reviews, ui_review, word_emb):
    idx_ur = user_reviews.reshape(-1, CHUNK)
    idx_irui = jnp.concatenate([
        item_reviews.reshape(-1, CHUNK),
        ui_review.reshape(-1, CHUNK),
    ])
    out_ur, out_ir, out_ui = _run(idx_ur, idx_irui, word_emb)
    return (
        out_ur.reshape(B, SENT_COUNT * SENT_LENGTH, DIM),
        out_ir.reshape(B, SENT_COUNT * SENT_LENGTH, DIM),
        out_ui.reshape(B, SENT_LENGTH, DIM),
    )


# speedup vs baseline: 1.0983x; 1.0003x over previous
import functools

import jax
import jax.numpy as jnp
from jax import lax
from jax.experimental import pallas as pl
from jax.experimental.pallas import tpu as pltpu
from jax.experimental.pallas import tpu_sc as plsc

VOCAB = 100000
DIM = 64
B = 1024
SENT_COUNT = 10
SENT_LENGTH = 20

N_UR = B * SENT_COUNT * SENT_LENGTH
N_UI = B * SENT_LENGTH

NC = 2
NS = 16
NW = NC * NS

CHUNK = 128
K = 5
GROUP = K * CHUNK

CH_UR = N_UR // NW // CHUNK
CH_UI = N_UI // NW // CHUNK
NG_UR = CH_UR // K

PW_UR = N_UR // NW
PW_UI = N_UI // NW


def _pipeline(table, idx_v, rows_v, gsems, ssems, wid, stores, ng):
    def fire_group(g, p):
        for b in range(K):
            pltpu.async_copy(
                table.at[idx_v.at[g * K + b]],
                rows_v.at[p, pl.ds(b * CHUNK, CHUNK)],
                gsems[p])

    def drain_gathers(p):
        pltpu.make_async_copy(table.at[pl.ds(0, GROUP)],
                              rows_v.at[p], gsems[p]).wait()

    fire_group(0, 0)
    if ng > 1:
        fire_group(1, 1)

    def body(i, carry):
        for p in (0, 1):
            g = 2 * i + p
            drain_gathers(p)
            stores(g, p)
            fire_group(g + 2, p)
        return carry

    lax.fori_loop(0, (ng - 2) // 2, body, 0)
    for p in (0, 1):
        g = ng - 2 + p
        drain_gathers(p)
        stores(g, p)
    return drain_gathers


def _emb_ur(idx_hbm, table, out_ur,
            idx_v, rows_v, gsem0, gsem1, ssem0, ssem1):
    wid = lax.axis_index("s") * NC + lax.axis_index("c")
    pltpu.sync_copy(idx_hbm.at[pl.ds(wid * CH_UR, CH_UR)], idx_v)

    def stores(g, p):
        pltpu.async_copy(
            rows_v.at[p],
            out_ur.at[pl.ds(wid * PW_UR + g * GROUP, GROUP)],
            (ssem0, ssem1)[p]).wait()

    _pipeline(table, idx_v, rows_v, (gsem0, gsem1), (ssem0, ssem1),
              wid, stores, NG_UR)


def _emb_ui(idx_hbm, table, out_ui,
            idx_v, rows_v, gsem0, gsem1, ssem0, ssem1):
    wid = lax.axis_index("s") * NC + lax.axis_index("c")
    pltpu.sync_copy(idx_hbm.at[pl.ds(wid * CH_UI, CH_UI)], idx_v)
    for b in range(K):
        pltpu.async_copy(
            table.at[idx_v.at[b]],
            rows_v.at[0, pl.ds(b * CHUNK, CHUNK)],
            gsem0)
    pltpu.make_async_copy(table.at[pl.ds(0, GROUP)],
                          rows_v.at[0], gsem0).wait()
    pltpu.sync_copy(rows_v.at[0], out_ui.at[pl.ds(wid * PW_UI, PW_UI)])


@jax.jit
def _run(idx_ur, idx_ir, idx_ui, word_emb):
    mesh = plsc.VectorSubcoreMesh(core_axis_name="c", subcore_axis_name="s")
    common = dict(
        mesh=mesh,
        compiler_params=pltpu.CompilerParams(use_tc_tiling_on_sc=False),
    )
    big_scratch = [
        pltpu.VMEM((CH_UR, CHUNK), jnp.int32),
        pltpu.VMEM((2, GROUP, DIM), jnp.float32),
        pltpu.SemaphoreType.DMA,
        pltpu.SemaphoreType.DMA,
        pltpu.SemaphoreType.DMA,
        pltpu.SemaphoreType.DMA,
    ]
    out_ur = pl.kernel(
        _emb_ur,
        out_type=jax.ShapeDtypeStruct((N_UR, DIM), jnp.float32),
        scratch_types=big_scratch,
        **common,
    )(idx_ur, word_emb)
    out_ir = pl.kernel(
        _emb_ur,
        out_type=jax.ShapeDtypeStruct((N_UR, DIM), jnp.float32),
        scratch_types=big_scratch,
        **common,
    )(idx_ir, word_emb)
    out_ui = pl.kernel(
        _emb_ui,
        out_type=jax.ShapeDtypeStruct((N_UI, DIM), jnp.float32),
        scratch_types=[
            pltpu.VMEM((CH_UI, CHUNK), jnp.int32),
            pltpu.VMEM((2, GROUP, DIM), jnp.float32),
            pltpu.SemaphoreType.DMA,
            pltpu.SemaphoreType.DMA,
            pltpu.SemaphoreType.DMA,
            pltpu.SemaphoreType.DMA,
        ],
        **common,
    )(idx_ui, word_emb)
    return out_ur, out_ir, out_ui


def kernel(user_reviews, item_reviews, ui_review, word_emb):
    idx_ur = user_reviews.reshape(-1, CHUNK)
    idx_ir = item_reviews.reshape(-1, CHUNK)
    idx_ui = ui_review.reshape(-1, CHUNK)
    out_ur, out_ir, out_ui = _run(idx_ur, idx_ir, idx_ui, word_emb)
    return (
        out_ur.reshape(B, SENT_COUNT * SENT_LENGTH, DIM),
        out_ir.reshape(B, SENT_COUNT * SENT_LENGTH, DIM),
        out_ui.reshape(B, SENT_LENGTH, DIM),
    )
